# baseline (speedup 1.0000x reference)
import dataclasses
import math

import numpy as np

import jax
import jax.numpy as jnp
from jax.experimental import pallas as pl
from jax.experimental.pallas import tpu as pltpu
from jax.experimental.pallas import tpu_sc as plsc

N = 4096
V = 32000
LS = 0.1
PAD = 0

_EPS = float(np.float32(LS / (V - 1)))
_P = 1.0 - LS
_C1 = _EPS * math.log(_EPS)
_C2 = _P * math.log(_P)

_ROW_BLK = 128
_GATHER_W = 128
_SC_ROWS = 512
_NT = N - _SC_ROWS
_N_SUB = 32


def _colsum_body(x_ref, y_ref, cs_ref, g_ref):
    i = pl.program_id(0)

    @pl.when(i == 0)
    def _init():
        cs_ref[...] = jnp.zeros_like(cs_ref)

    x = x_ref[...]
    parts = [x[8 * k:8 * (k + 1)] for k in range(_ROW_BLK // 8)]
    while len(parts) > 1:
        parts = [parts[j] + parts[j + 1] for j in range(0, len(parts), 2)]
    cs_ref[...] += parts[0]
    cols = jax.lax.broadcasted_iota(jnp.int32, (_ROW_BLK, V), 1)
    yv = jnp.transpose(y_ref[0:1, pl.ds(i * _ROW_BLK, _ROW_BLK)])
    g_ref[pl.ds(i * _ROW_BLK, _ROW_BLK), :] = jnp.sum(
        jnp.where(cols == yv, x, jnp.float32(0.0)),
        axis=1, keepdims=True)


def _combine_body(cs_ref, y_ref, yy_ref, g_ref, sccols_ref, scs_ref,
                  scg_ref, o_ref):
    yv = y_ref[...]
    yyv = yy_ref[...]
    cs8 = cs_ref[...]
    word = jnp.sum((yv != PAD).astype(jnp.float32))
    m = jnp.float32(N) - word
    masked = (yv < N) & (yyv == PAD)
    K = jnp.float32(N) - jnp.sum(masked.astype(jnp.float32))
    u = jnp.where(masked, jnp.float32(0.0), jnp.float32(1.0))
    G = (jax.lax.dot_general(u[:, 0:_NT], g_ref[...],
                             (((1,), (0,)), ((), ())),
                             preferred_element_type=jnp.float32)[0, 0]
         + jnp.sum(scg_ref[:, 0:1]))
    S_total = jnp.sum(cs8) + jnp.sum(scs_ref[...])
    cs_first = (jnp.sum(cs8[:, 0:N], axis=0, keepdims=True)
                + jnp.sum(sccols_ref[...], axis=0, keepdims=True))
    S_masked = jnp.sum(jnp.where(yv == PAD, cs_first, jnp.float32(0.0)))
    kl = (jnp.float32(N) * (jnp.float32(V) - m) * jnp.float32(_C1)
          + K * jnp.float32(_C2 - _C1)
          - jnp.float32(_EPS) * (S_total - S_masked)
          - jnp.float32(_P - _EPS) * G)
    o_ref[...] = (kl / word)[None, None]


def _sc_gather_yy(y_tbl, y_idx):
    mesh = plsc.VectorSubcoreMesh(core_axis_name="c", subcore_axis_name="s")
    cp = pltpu.CompilerParams()
    if "needs_layout_passes" in pltpu.CompilerParams.__dataclass_fields__:
        cp = dataclasses.replace(cp, needs_layout_passes=False)

    @pl.kernel(
        out_type=jax.ShapeDtypeStruct((1, N), jnp.int32),
        mesh=mesh,
        compiler_params=cp,
    )
    def run(ytbl_hbm, yidx_hbm, yy_hbm):
        def body(ytbl_vmem, yc_vmem, yy_vmem):
            @pl.loop(0, _GATHER_W, step=16)
            def _(k):
                idx = jnp.minimum(yc_vmem[0, pl.ds(k, 16)], N - 1)
                vals = plsc.load_gather(ytbl_vmem, [jnp.zeros_like(idx), idx])
                yy_vmem[0, pl.ds(k, 16)] = vals

        pltpu.emit_pipeline(
            body,
            grid=(N // _GATHER_W,),
            in_specs=[
                pl.BlockSpec((1, N), lambda i: (0, 0)),
                pl.BlockSpec((1, _GATHER_W), lambda i: (0, i)),
            ],
            out_specs=[pl.BlockSpec((1, _GATHER_W), lambda i: (0, i))],
            core_axis_name=("c", "s"),
            dimension_semantics=(pltpu.PARALLEL,),
        )(ytbl_hbm, yidx_hbm, yy_hbm)

    return run(y_tbl, y_idx)


def _sc_stream(out2, y_tbl, y_sc):
    mesh = plsc.VectorSubcoreMesh(core_axis_name="c", subcore_axis_name="s")
    cp = pltpu.CompilerParams()
    if "needs_layout_passes" in pltpu.CompilerParams.__dataclass_fields__:
        cp = dataclasses.replace(cp, needs_layout_passes=False)

    @pl.kernel(
        out_type=(
            jax.ShapeDtypeStruct((_N_SUB, N), jnp.float32),
            jax.ShapeDtypeStruct((_N_SUB, 16), jnp.float32),
            jax.ShapeDtypeStruct((_N_SUB, 16), jnp.float32),
        ),
        mesh=mesh,
        compiler_params=cp,
        scratch_types=[
            pltpu.VMEM((N,), jnp.float32),
            pltpu.VMEM((16,), jnp.float32),
            pltpu.VMEM((16,), jnp.float32),
            pltpu.SemaphoreType.DMA,
        ],
    )
    def run(out_hbm, ytbl_hbm, ysc_hbm, cols_hbm, s_hbm, g_hbm,
            colacc, svec, gacc, sem):
        sub = jax.lax.axis_index("c") * 16 + jax.lax.axis_index("s")
        zeros16 = jnp.zeros((16,), jnp.float32)
        svec[...] = zeros16
        gacc[...] = zeros16

        @pl.loop(0, N, step=16)
        def _(c):
            colacc[pl.ds(c, 16)] = zeros16

        def body(ytbl_vmem, ysc_vmem, row_vmem):
            @pl.loop(0, N, step=256)
            def _(c):
                regs = [row_vmem[0, pl.ds(c + 16 * j, 16)] for j in range(16)]
                for j in range(16):
                    plsc.addupdate(colacc.at[pl.ds(c + 16 * j, 16)], regs[j])
                while len(regs) > 1:
                    regs = [regs[k] + regs[k + 1] for k in range(0, len(regs), 2)]
                plsc.addupdate(svec.at[pl.ds(0, 16)], regs[0])

            @pl.loop(N, V, step=256)
            def _(c):
                regs = [row_vmem[0, pl.ds(c + 16 * j, 16)] for j in range(16)]
                while len(regs) > 1:
                    regs = [regs[k] + regs[k + 1] for k in range(0, len(regs), 2)]
                plsc.addupdate(svec.at[pl.ds(0, 16)], regs[0])

            yv16 = ysc_vmem[0, pl.ds(0, 16)]
            zeros_i = jnp.zeros_like(yv16)
            yy16 = plsc.load_gather(ytbl_vmem,
                                    [zeros_i, jnp.minimum(yv16, N - 1)])
            gv16 = plsc.load_gather(row_vmem, [zeros_i, yv16])
            unmasked = (yv16 >= N) | (yy16 != PAD)
            plsc.addupdate(gacc.at[pl.ds(0, 16)],
                           jnp.where(unmasked, gv16, jnp.float32(0.0)))

        pltpu.emit_pipeline(
            body,
            grid=(_SC_ROWS,),
            in_specs=[
                pl.BlockSpec((1, N), lambda i: (0, 0)),
                pl.BlockSpec((1, 16), lambda i: (i, 0)),
                pl.BlockSpec((1, V), lambda i: (i + _NT, 0)),
            ],
            out_specs=[],
            core_axis_name=("c", "s"),
            dimension_semantics=(pltpu.PARALLEL,),
        )(ytbl_hbm, ysc_hbm, out_hbm)

        pltpu.async_copy(colacc, cols_hbm.at[sub], sem).wait()
        pltpu.async_copy(svec, s_hbm.at[sub], sem).wait()
        pltpu.async_copy(gacc, g_hbm.at[sub], sem).wait()

    return run(out2, y_tbl, y_sc)


def kernel(out, y):
    y = y.reshape(-1).astype(jnp.int32)
    out2 = out.reshape(N, V)
    y_row = y.reshape(1, N)

    yy = _sc_gather_yy(y_row, y_row)

    sc_cols, sc_s, sc_g = _sc_stream(
        out2, y_row,
        jnp.broadcast_to(y[_NT:, None], (_SC_ROWS, 16)))

    cs8, g = pl.pallas_call(
        _colsum_body,
        grid=(_NT // _ROW_BLK,),
        in_specs=[
            pl.BlockSpec((_ROW_BLK, V), lambda i: (i, 0)),
            pl.BlockSpec((1, N), lambda i: (0, 0)),
        ],
        out_specs=[
            pl.BlockSpec((8, V), lambda i: (0, 0)),
            pl.BlockSpec((_NT, 1), lambda i: (0, 0)),
        ],
        out_shape=[
            jax.ShapeDtypeStruct((8, V), jnp.float32),
            jax.ShapeDtypeStruct((_NT, 1), jnp.float32),
        ],
    )(out2, y_row)

    res = pl.pallas_call(
        _combine_body,
        in_specs=[
            pl.BlockSpec((8, V), lambda: (0, 0)),
            pl.BlockSpec((1, N), lambda: (0, 0)),
            pl.BlockSpec((1, N), lambda: (0, 0)),
            pl.BlockSpec((_NT, 1), lambda: (0, 0)),
            pl.BlockSpec((_N_SUB, N), lambda: (0, 0)),
            pl.BlockSpec((_N_SUB, 16), lambda: (0, 0)),
            pl.BlockSpec((_N_SUB, 16), lambda: (0, 0)),
        ],
        out_specs=pl.BlockSpec((1, 1), lambda: (0, 0)),
        out_shape=jax.ShapeDtypeStruct((1, 1), jnp.float32),
    )(cs8, y_row, yy, g, sc_cols, sc_s, sc_g)

    return res[0, 0]


# speedup vs baseline: 1.0282x; 1.0282x over previous
import dataclasses
import math

import numpy as np

import jax
import jax.numpy as jnp
from jax.experimental import pallas as pl
from jax.experimental.pallas import tpu as pltpu
from jax.experimental.pallas import tpu_sc as plsc

N = 4096
V = 32000
LS = 0.1
PAD = 0

_EPS = float(np.float32(LS / (V - 1)))
_P = 1.0 - LS
_C1 = _EPS * math.log(_EPS)
_C2 = _P * math.log(_P)

_ROW_BLK = 128
_GATHER_W = 128


def _colsum_body(x_ref, y_ref, cs_ref, g_ref):
    i = pl.program_id(0)

    @pl.when(i == 0)
    def _init():
        cs_ref[...] = jnp.zeros_like(cs_ref)

    x = x_ref[...]
    parts = [x[8 * k:8 * (k + 1)] for k in range(_ROW_BLK // 8)]
    while len(parts) > 1:
        parts = [parts[j] + parts[j + 1] for j in range(0, len(parts), 2)]
    cs_ref[...] += parts[0]
    cols = jax.lax.broadcasted_iota(jnp.int32, (_ROW_BLK, V), 1)
    yv = jnp.transpose(y_ref[0:1, pl.ds(i * _ROW_BLK, _ROW_BLK)])
    g_ref[pl.ds(i * _ROW_BLK, _ROW_BLK), :] = jnp.sum(
        jnp.where(cols == yv, x, jnp.float32(0.0)),
        axis=1, keepdims=True)


def _combine_body(cs_ref, y_ref, yy_ref, g_ref, o_ref):
    yv = y_ref[...]
    yyv = yy_ref[...]
    cs8 = cs_ref[...]
    word = jnp.sum((yv != PAD).astype(jnp.float32))
    m = jnp.float32(N) - word
    masked = (yv < N) & (yyv == PAD)
    K = jnp.float32(N) - jnp.sum(masked.astype(jnp.float32))
    u = jnp.where(masked, jnp.float32(0.0), jnp.float32(1.0))
    G = jax.lax.dot_general(u, g_ref[...], (((1,), (0,)), ((), ())),
                            preferred_element_type=jnp.float32)[0, 0]
    S_total = jnp.sum(cs8)
    cs_first = jnp.sum(cs8[:, 0:N], axis=0, keepdims=True)
    S_masked = jnp.sum(jnp.where(yv == PAD, cs_first, jnp.float32(0.0)))
    kl = (jnp.float32(N) * (jnp.float32(V) - m) * jnp.float32(_C1)
          + K * jnp.float32(_C2 - _C1)
          - jnp.float32(_EPS) * (S_total - S_masked)
          - jnp.float32(_P - _EPS) * G)
    o_ref[...] = (kl / word)[None, None]


def _sc_gather_yy(y_tbl, y_idx):
    mesh = plsc.VectorSubcoreMesh(core_axis_name="c", subcore_axis_name="s")
    cp = pltpu.CompilerParams()
    if "needs_layout_passes" in pltpu.CompilerParams.__dataclass_fields__:
        cp = dataclasses.replace(cp, needs_layout_passes=False)

    @pl.kernel(
        out_type=jax.ShapeDtypeStruct((1, N), jnp.int32),
        mesh=mesh,
        compiler_params=cp,
    )
    def run(ytbl_hbm, yidx_hbm, yy_hbm):
        def body(ytbl_vmem, yc_vmem, yy_vmem):
            @pl.loop(0, _GATHER_W, step=16)
            def _(k):
                idx = jnp.minimum(yc_vmem[0, pl.ds(k, 16)], N - 1)
                vals = plsc.load_gather(ytbl_vmem, [jnp.zeros_like(idx), idx])
                yy_vmem[0, pl.ds(k, 16)] = vals

        pltpu.emit_pipeline(
            body,
            grid=(N // _GATHER_W,),
            in_specs=[
                pl.BlockSpec((1, N), lambda i: (0, 0)),
                pl.BlockSpec((1, _GATHER_W), lambda i: (0, i)),
            ],
            out_specs=[pl.BlockSpec((1, _GATHER_W), lambda i: (0, i))],
            core_axis_name=("c", "s"),
            dimension_semantics=(pltpu.PARALLEL,),
        )(ytbl_hbm, yidx_hbm, yy_hbm)

    return run(y_tbl, y_idx)


def kernel(out, y):
    y = y.reshape(-1).astype(jnp.int32)
    out2 = out.reshape(N, V)
    y_row = y.reshape(1, N)

    yy = _sc_gather_yy(y_row, y_row)

    cs8, g = pl.pallas_call(
        _colsum_body,
        grid=(N // _ROW_BLK,),
        in_specs=[
            pl.BlockSpec((_ROW_BLK, V), lambda i: (i, 0)),
            pl.BlockSpec((1, N), lambda i: (0, 0)),
        ],
        out_specs=[
            pl.BlockSpec((8, V), lambda i: (0, 0)),
            pl.BlockSpec((N, 1), lambda i: (0, 0)),
        ],
        out_shape=[
            jax.ShapeDtypeStruct((8, V), jnp.float32),
            jax.ShapeDtypeStruct((N, 1), jnp.float32),
        ],
    )(out2, y_row)

    res = pl.pallas_call(
        _combine_body,
        in_specs=[
            pl.BlockSpec((8, V), lambda: (0, 0)),
            pl.BlockSpec((1, N), lambda: (0, 0)),
            pl.BlockSpec((1, N), lambda: (0, 0)),
            pl.BlockSpec((N, 1), lambda: (0, 0)),
        ],
        out_specs=pl.BlockSpec((1, 1), lambda: (0, 0)),
        out_shape=jax.ShapeDtypeStruct((1, 1), jnp.float32),
    )(cs8, y_row, yy, g)

    return res[0, 0]
